# Initial kernel scaffold; baseline (speedup 1.0000x reference)
import jax
import jax.numpy as jnp
from jax.experimental import pallas as pl


def kernel(points, embeddings, instance_labels):
    raise NotImplementedError("write your pallas kernel here")


# speedup vs baseline: 4.5331x; 4.5331x over previous
import functools

import jax
import jax.numpy as jnp
from jax import lax
from jax.experimental import pallas as pl

_DELTA_V = 0.1
_DELTA_D = 0.5
_K = 20
_NUM_IDS = 20
_BIG = 1e30


def _normalize_body(e_ref, o_ref):
    e = e_ref[0]
    n = jnp.sqrt(jnp.sum(e * e, axis=1, keepdims=True))
    o_ref[0] = e / jnp.maximum(n, 1e-12)


def _normalize(emb):
    B, N, D = emb.shape
    return pl.pallas_call(
        _normalize_body,
        grid=(B,),
        in_specs=[pl.BlockSpec((1, N, D), lambda b: (b, 0, 0))],
        out_specs=pl.BlockSpec((1, N, D), lambda b: (b, 0, 0)),
        out_shape=jax.ShapeDtypeStruct((B, N, D), jnp.float32),
    )(emb)


def _smooth_body(p_ref, pa_ref, sr_ref, eb_ref, ea_ref, o_ref, *, R, N, K):
    r = pl.program_id(1)
    first = (pl.program_id(0) == 0) & (r == 0)

    p_blk = p_ref[0]
    p_all = pa_ref[0]
    s_row = sr_ref[0]
    e_blk = eb_ref[0]
    e_all = ea_ref[0]

    dot = lax.dot_general(p_blk, p_all, (((1,), (1,)), ((), ())),
                          preferred_element_type=jnp.float32)
    s_col = jnp.sum(p_blk * p_blk, axis=1, keepdims=True)
    d2 = (s_col + s_row) - 2.0 * dot
    key = jnp.where(d2 > 1e-12, d2, 0.0)

    g = lax.dot_general(e_blk, e_all, (((1,), (1,)), ((), ())),
                        preferred_element_type=jnp.float32,
                        precision=lax.Precision.HIGHEST)
    hg = jnp.sqrt(jnp.maximum(2.0 - 2.0 * g, 0.0))

    col = lax.broadcasted_iota(jnp.int32, (R, N), 1)

    def body(t, carry):
        keyc, s = carry
        m = jnp.min(keyc, axis=1, keepdims=True)
        cand = jnp.where(keyc == m, col, N)
        j = jnp.min(cand, axis=1, keepdims=True)
        sel = col == j
        w = jnp.where(t > 0, 1.0, 0.0)
        s = s + w * jnp.sum(jnp.where(sel, hg, 0.0))
        keyc = jnp.where(sel, _BIG, keyc)
        return keyc, s

    _, s = lax.fori_loop(0, K + 1, body, (key, jnp.float32(0.0)))

    @pl.when(first)
    def _():
        o_ref[...] = jnp.zeros((1, 1), jnp.float32)

    o_ref[...] += jnp.reshape(s, (1, 1))


def _smooth(pts_pad, s_row, en, R):
    B, N, _ = pts_pad.shape
    D = en.shape[2]
    NR = N // R
    return pl.pallas_call(
        functools.partial(_smooth_body, R=R, N=N, K=_K),
        grid=(B, NR),
        in_specs=[
            pl.BlockSpec((1, R, 8), lambda b, r: (b, r, 0)),
            pl.BlockSpec((1, N, 8), lambda b, r: (b, 0, 0)),
            pl.BlockSpec((1, 1, N), lambda b, r: (b, 0, 0)),
            pl.BlockSpec((1, R, D), lambda b, r: (b, r, 0)),
            pl.BlockSpec((1, N, D), lambda b, r: (b, 0, 0)),
        ],
        out_specs=pl.BlockSpec((1, 1), lambda b, r: (0, 0)),
        out_shape=jax.ShapeDtypeStruct((1, 1), jnp.float32),
    )(pts_pad, pts_pad, s_row, en, en)


def _instance_body(lab_ref, en_ref, ent_ref, pull_ref, push_ref, *, S, N, D):
    first = pl.program_id(0) == 0
    lab = lab_ref[0]
    en = en_ref[0]
    ent = ent_ref[0]

    seg = lax.broadcasted_iota(jnp.int32, (S, N), 0)
    oh = (seg == lab).astype(jnp.float32)
    cnt = jnp.sum(oh, axis=1, keepdims=True)
    sums = lax.dot_general(oh, en, (((1,), (0,)), ((), ())),
                           preferred_element_type=jnp.float32, precision=lax.Precision.HIGHEST)
    den = jnp.where(cnt > 0, cnt, 1.0)
    means = sums / den

    e2row = jnp.sum(ent * ent, axis=0, keepdims=True)
    x = lax.dot_general(means, ent, (((1,), (0,)), ((), ())),
                        preferred_element_type=jnp.float32, precision=lax.Precision.HIGHEST)
    m2col = jnp.sum(means * means, axis=1, keepdims=True)
    d2 = e2row - 2.0 * x + m2col
    pos = d2 > 1e-12
    d = jnp.where(pos, jnp.sqrt(jnp.where(pos, d2, 1.0)), 0.0)
    d = jnp.maximum(d - _DELTA_V, 0.0)
    segsum = jnp.sum(d * oh, axis=1, keepdims=True)

    rid = lax.broadcasted_iota(jnp.int32, (S, 1), 0)
    valid = (rid >= 1) & (rid < _NUM_IDS)
    prescol = valid & (cnt > 0)
    ipl = jnp.sum(jnp.where(prescol, segsum / den, 0.0))
    ni = jnp.sum(prescol.astype(jnp.float32))
    pull_b = ipl / (ni + 1e-6)

    mn = means / jnp.maximum(jnp.sqrt(m2col), 1e-12)
    mm = lax.dot_general(mn, mn, (((1,), (1,)), ((), ())),
                         preferred_element_type=jnp.float32, precision=lax.Precision.HIGHEST)
    ri = lax.broadcasted_iota(jnp.int32, (S, S), 0)
    ci = lax.broadcasted_iota(jnp.int32, (S, S), 1)
    diag = ri == ci
    mn2row = jnp.sum(jnp.where(diag, mm, 0.0), axis=0, keepdims=True)
    mn2col = jnp.sum(jnp.where(diag, mm, 0.0), axis=1, keepdims=True)
    d2m = mn2row + mn2col - 2.0 * mm
    posm = d2m > 1e-12
    dm = jnp.where(posm, jnp.sqrt(jnp.where(posm, d2m, 1.0)), 0.0)
    presf = prescol.astype(jnp.float32)
    presrow = jnp.sum(jnp.where(diag, presf, 0.0), axis=0, keepdims=True)
    tri = (ci > ri).astype(jnp.float32) * presf * presrow
    hinge = jnp.maximum(2.0 * _DELTA_D - dm, 0.0)
    push_b = jnp.sum(hinge * tri) / (jnp.sum(tri) + 1e-6)

    @pl.when(first)
    def _():
        pull_ref[...] = jnp.zeros((1, 1), jnp.float32)
        push_ref[...] = jnp.zeros((1, 1), jnp.float32)

    pull_ref[...] += jnp.reshape(pull_b, (1, 1))
    push_ref[...] += jnp.reshape(push_b, (1, 1))


def _instance(labels, en, ent):
    B, N, D = en.shape
    S = 32
    return pl.pallas_call(
        functools.partial(_instance_body, S=S, N=N, D=D),
        grid=(B,),
        in_specs=[
            pl.BlockSpec((1, 1, N), lambda b: (b, 0, 0)),
            pl.BlockSpec((1, N, D), lambda b: (b, 0, 0)),
            pl.BlockSpec((1, D, N), lambda b: (b, 0, 0)),
        ],
        out_specs=[
            pl.BlockSpec((1, 1), lambda b: (0, 0)),
            pl.BlockSpec((1, 1), lambda b: (0, 0)),
        ],
        out_shape=[
            jax.ShapeDtypeStruct((1, 1), jnp.float32),
            jax.ShapeDtypeStruct((1, 1), jnp.float32),
        ],
    )(labels, en, ent)


def kernel(points, embeddings, instance_labels):
    B, N, D = embeddings.shape
    pts = points.astype(jnp.float32)
    lab = instance_labels.astype(jnp.int32).reshape(B, 1, N)

    pts_pad = jnp.pad(pts, ((0, 0), (0, 0), (0, 8 - pts.shape[2])))
    s_row = jnp.sum(pts * pts, axis=2)[:, None, :]

    en = _normalize(embeddings.astype(jnp.float32))
    ent = jnp.transpose(en, (0, 2, 1))

    R = 128
    sm_sum = _smooth(pts_pad, s_row, en, R)[0, 0]
    pull_s, push_s = _instance(lab, en, ent)

    sm = sm_sum / jnp.float32(N * _K * B)
    pull = pull_s[0, 0] / jnp.float32(B)
    push = push_s[0, 0] / jnp.float32(B)
    inst = pull + push
    total = inst + 0.1 * sm
    return total, inst, sm, pull, push


# speedup vs baseline: 13.0306x; 2.8745x over previous
import functools

import jax
import jax.numpy as jnp
from jax import lax
from jax.experimental import pallas as pl

_DELTA_V = 0.1
_DELTA_D = 0.5
_K = 20
_NUM_IDS = 20
_BIG = 1e30


def _normalize_body(e_ref, o_ref):
    e = e_ref[0]
    n = jnp.sqrt(jnp.sum(e * e, axis=1, keepdims=True))
    o_ref[0] = e / jnp.maximum(n, 1e-12)


def _normalize(emb):
    B, N, D = emb.shape
    return pl.pallas_call(
        _normalize_body,
        grid=(B,),
        in_specs=[pl.BlockSpec((1, N, D), lambda b: (b, 0, 0))],
        out_specs=pl.BlockSpec((1, N, D), lambda b: (b, 0, 0)),
        out_shape=jax.ShapeDtypeStruct((B, N, D), jnp.float32),
    )(emb)


def _smooth_body(p_ref, pa_ref, sr_ref, eb_ref, ea_ref, o_ref, *, R, N, K):
    r = pl.program_id(1)
    first = (pl.program_id(0) == 0) & (r == 0)

    p_blk = p_ref[0]
    p_all = pa_ref[0]
    s_row = sr_ref[0]
    e_blk = eb_ref[0]
    e_all = ea_ref[0]

    dot = lax.dot_general(p_blk, p_all, (((1,), (1,)), ((), ())),
                          preferred_element_type=jnp.float32)
    s_col = jnp.sum(p_blk * p_blk, axis=1, keepdims=True)
    d2 = (s_col + s_row) - 2.0 * dot
    key = jnp.where(d2 > 1e-12, d2, 0.0)

    g = lax.dot_general(e_blk, e_all, (((1,), (1,)), ((), ())),
                        preferred_element_type=jnp.float32,
                        precision=lax.Precision.HIGHEST)
    hg = jnp.sqrt(jnp.maximum(2.0 - 2.0 * g, 0.0))

    col = lax.broadcasted_iota(jnp.int32, (R, N), 1)

    zf = key == 0.0
    zn = jnp.sum(zf.astype(jnp.float32), axis=1, keepdims=True)
    s0_all = jnp.sum(jnp.where(zf, hg, 0.0), axis=1, keepdims=True)
    fz = jnp.min(jnp.where(zf, col, N), axis=1, keepdims=True)
    hg_fz = jnp.sum(jnp.where(col == fz, hg, 0.0), axis=1, keepdims=True)
    s_zero = s0_all - hg_fz

    pkey = jnp.where(zf, _BIG, key)
    m0 = jnp.min(pkey, axis=1, keepdims=True)
    fp = jnp.min(jnp.where(pkey == m0, col, N), axis=1, keepdims=True)
    hg_fp = jnp.sum(jnp.where(col == fp, hg, 0.0), axis=1, keepdims=True)

    def body(t, carry):
        m, t_star = carry
        tf = t.astype(jnp.float32)
        t_star = jnp.where(tf == 20.0 - zn, m, t_star)
        nxt = jnp.min(jnp.where(pkey > m, pkey, _BIG), axis=1, keepdims=True)
        return nxt, t_star

    _, t_star = lax.fori_loop(
        0, K + 1, body, (m0, jnp.zeros((R, 1), jnp.float32) - 1.0))

    s_pos = jnp.sum(jnp.where(pkey <= t_star, hg, 0.0))
    s = (s_pos + jnp.sum(s_zero)
         - jnp.sum(jnp.where(zn == 0.0, hg_fp, 0.0)))

    @pl.when(first)
    def _():
        o_ref[...] = jnp.zeros((1, 1), jnp.float32)

    o_ref[...] += jnp.reshape(s, (1, 1))


def _smooth(pts_pad, s_row, en, R):
    B, N, _ = pts_pad.shape
    D = en.shape[2]
    NR = N // R
    return pl.pallas_call(
        functools.partial(_smooth_body, R=R, N=N, K=_K),
        grid=(B, NR),
        in_specs=[
            pl.BlockSpec((1, R, 8), lambda b, r: (b, r, 0)),
            pl.BlockSpec((1, N, 8), lambda b, r: (b, 0, 0)),
            pl.BlockSpec((1, 1, N), lambda b, r: (b, 0, 0)),
            pl.BlockSpec((1, R, D), lambda b, r: (b, r, 0)),
            pl.BlockSpec((1, N, D), lambda b, r: (b, 0, 0)),
        ],
        out_specs=pl.BlockSpec((1, 1), lambda b, r: (0, 0)),
        out_shape=jax.ShapeDtypeStruct((1, 1), jnp.float32),
    )(pts_pad, pts_pad, s_row, en, en)


def _instance_body(lab_ref, en_ref, ent_ref, pull_ref, push_ref, *, S, N, D):
    first = pl.program_id(0) == 0
    lab = lab_ref[0]
    en = en_ref[0]
    ent = ent_ref[0]

    seg = lax.broadcasted_iota(jnp.int32, (S, N), 0)
    oh = (seg == lab).astype(jnp.float32)
    cnt = jnp.sum(oh, axis=1, keepdims=True)
    sums = lax.dot_general(oh, en, (((1,), (0,)), ((), ())),
                           preferred_element_type=jnp.float32, precision=lax.Precision.HIGHEST)
    den = jnp.where(cnt > 0, cnt, 1.0)
    means = sums / den

    e2row = jnp.sum(ent * ent, axis=0, keepdims=True)
    x = lax.dot_general(means, ent, (((1,), (0,)), ((), ())),
                        preferred_element_type=jnp.float32, precision=lax.Precision.HIGHEST)
    m2col = jnp.sum(means * means, axis=1, keepdims=True)
    d2 = e2row - 2.0 * x + m2col
    pos = d2 > 1e-12
    d = jnp.where(pos, jnp.sqrt(jnp.where(pos, d2, 1.0)), 0.0)
    d = jnp.maximum(d - _DELTA_V, 0.0)
    segsum = jnp.sum(d * oh, axis=1, keepdims=True)

    rid = lax.broadcasted_iota(jnp.int32, (S, 1), 0)
    valid = (rid >= 1) & (rid < _NUM_IDS)
    prescol = valid & (cnt > 0)
    ipl = jnp.sum(jnp.where(prescol, segsum / den, 0.0))
    ni = jnp.sum(prescol.astype(jnp.float32))
    pull_b = ipl / (ni + 1e-6)

    mn = means / jnp.maximum(jnp.sqrt(m2col), 1e-12)
    mm = lax.dot_general(mn, mn, (((1,), (1,)), ((), ())),
                         preferred_element_type=jnp.float32, precision=lax.Precision.HIGHEST)
    ri = lax.broadcasted_iota(jnp.int32, (S, S), 0)
    ci = lax.broadcasted_iota(jnp.int32, (S, S), 1)
    diag = ri == ci
    mn2row = jnp.sum(jnp.where(diag, mm, 0.0), axis=0, keepdims=True)
    mn2col = jnp.sum(jnp.where(diag, mm, 0.0), axis=1, keepdims=True)
    d2m = mn2row + mn2col - 2.0 * mm
    posm = d2m > 1e-12
    dm = jnp.where(posm, jnp.sqrt(jnp.where(posm, d2m, 1.0)), 0.0)
    presf = prescol.astype(jnp.float32)
    presrow = jnp.sum(jnp.where(diag, presf, 0.0), axis=0, keepdims=True)
    tri = (ci > ri).astype(jnp.float32) * presf * presrow
    hinge = jnp.maximum(2.0 * _DELTA_D - dm, 0.0)
    push_b = jnp.sum(hinge * tri) / (jnp.sum(tri) + 1e-6)

    @pl.when(first)
    def _():
        pull_ref[...] = jnp.zeros((1, 1), jnp.float32)
        push_ref[...] = jnp.zeros((1, 1), jnp.float32)

    pull_ref[...] += jnp.reshape(pull_b, (1, 1))
    push_ref[...] += jnp.reshape(push_b, (1, 1))


def _instance(labels, en, ent):
    B, N, D = en.shape
    S = 32
    return pl.pallas_call(
        functools.partial(_instance_body, S=S, N=N, D=D),
        grid=(B,),
        in_specs=[
            pl.BlockSpec((1, 1, N), lambda b: (b, 0, 0)),
            pl.BlockSpec((1, N, D), lambda b: (b, 0, 0)),
            pl.BlockSpec((1, D, N), lambda b: (b, 0, 0)),
        ],
        out_specs=[
            pl.BlockSpec((1, 1), lambda b: (0, 0)),
            pl.BlockSpec((1, 1), lambda b: (0, 0)),
        ],
        out_shape=[
            jax.ShapeDtypeStruct((1, 1), jnp.float32),
            jax.ShapeDtypeStruct((1, 1), jnp.float32),
        ],
    )(labels, en, ent)


def kernel(points, embeddings, instance_labels):
    B, N, D = embeddings.shape
    pts = points.astype(jnp.float32)
    lab = instance_labels.astype(jnp.int32).reshape(B, 1, N)

    pts_pad = jnp.pad(pts, ((0, 0), (0, 0), (0, 8 - pts.shape[2])))
    s_row = jnp.sum(pts * pts, axis=2)[:, None, :]

    en = _normalize(embeddings.astype(jnp.float32))
    ent = jnp.transpose(en, (0, 2, 1))

    R = 128
    sm_sum = _smooth(pts_pad, s_row, en, R)[0, 0]
    pull_s, push_s = _instance(lab, en, ent)

    sm = sm_sum / jnp.float32(N * _K * B)
    pull = pull_s[0, 0] / jnp.float32(B)
    push = push_s[0, 0] / jnp.float32(B)
    inst = pull + push
    total = inst + 0.1 * sm
    return total, inst, sm, pull, push


# speedup vs baseline: 18.6402x; 1.4305x over previous
import functools

import jax
import jax.numpy as jnp
from jax import lax
from jax.experimental import pallas as pl
from jax.experimental.pallas import tpu as pltpu
from jax.experimental.pallas import tpu_sc as plsc

_DELTA_V = 0.1
_DELTA_D = 0.5
_K = 20
_NUM_IDS = 20
_BIG = 1e30


def _normalize_body(e_ref, o_ref):
    e = e_ref[0]
    n = jnp.sqrt(jnp.sum(e * e, axis=1, keepdims=True))
    o_ref[0] = e / jnp.maximum(n, 1e-12)


def _normalize(emb):
    B, N, D = emb.shape
    return pl.pallas_call(
        _normalize_body,
        grid=(B,),
        in_specs=[pl.BlockSpec((1, N, D), lambda b: (b, 0, 0))],
        out_specs=pl.BlockSpec((1, N, D), lambda b: (b, 0, 0)),
        out_shape=jax.ShapeDtypeStruct((B, N, D), jnp.float32),
    )(emb)


def _smooth_body(p_ref, pa_ref, sr_ref, eb_ref, ea_ref, o_ref, *, R, N, K):
    r = pl.program_id(1)
    first = (pl.program_id(0) == 0) & (r == 0)

    p_blk = p_ref[0]
    p_all = pa_ref[0]
    s_row = sr_ref[0]
    e_blk = eb_ref[0]
    e_all = ea_ref[0]

    dot = lax.dot_general(p_blk, p_all, (((1,), (1,)), ((), ())),
                          preferred_element_type=jnp.float32)
    s_col = jnp.sum(p_blk * p_blk, axis=1, keepdims=True)
    d2 = (s_col + s_row) - 2.0 * dot

    g = lax.dot_general(e_blk, e_all, (((1,), (1,)), ((), ())),
                        preferred_element_type=jnp.float32,
                        precision=lax.Precision.HIGHEST)
    hg = jnp.sqrt(jnp.maximum(2.0 - 2.0 * g, 0.0))

    colf = lax.broadcasted_iota(jnp.int32, (R, N), 1).astype(jnp.float32)
    keyp = jnp.where(d2 > 1e-12, d2, (colf + 1.0) * 1e-20)

    m0 = jnp.min(keyp, axis=1, keepdims=True)
    hg0 = jnp.sum(jnp.where(keyp == m0, hg, 0.0))

    def body(_, m):
        return jnp.min(jnp.where(keyp > m, keyp, _BIG), axis=1, keepdims=True)

    t_star = lax.fori_loop(0, K, body, m0)
    s = jnp.sum(jnp.where(keyp <= t_star, hg, 0.0)) - hg0

    @pl.when(first)
    def _():
        o_ref[...] = jnp.zeros((1, 1), jnp.float32)

    o_ref[...] += jnp.reshape(s, (1, 1))


def _smooth(pts_pad, s_row, en, R):
    B, N, _ = pts_pad.shape
    D = en.shape[2]
    NR = N // R
    return pl.pallas_call(
        functools.partial(_smooth_body, R=R, N=N, K=_K),
        grid=(B, NR),
        in_specs=[
            pl.BlockSpec((1, R, 8), lambda b, r: (b, r, 0)),
            pl.BlockSpec((1, N, 8), lambda b, r: (b, 0, 0)),
            pl.BlockSpec((1, 1, N), lambda b, r: (b, 0, 0)),
            pl.BlockSpec((1, R, D), lambda b, r: (b, r, 0)),
            pl.BlockSpec((1, N, D), lambda b, r: (b, 0, 0)),
        ],
        out_specs=pl.BlockSpec((1, 1), lambda b, r: (0, 0)),
        out_shape=jax.ShapeDtypeStruct((1, 1), jnp.float32),
    )(pts_pad, pts_pad, s_row, en, en)


def _seg_sums(eaug, labf, B, N, W):
    NW = 32
    CH = (B * N) // NW
    mesh = plsc.VectorSubcoreMesh(core_axis_name="c", subcore_axis_name="s")

    @functools.partial(
        pl.kernel, mesh=mesh,
        out_type=jax.ShapeDtypeStruct((NW * 32, W), jnp.float32),
        scratch_types=[pltpu.VMEM((CH,), jnp.int32),
                       pltpu.VMEM((128,), jnp.int32),
                       pltpu.VMEM((128,), jnp.int32),
                       pltpu.VMEM((128,), jnp.int32),
                       pltpu.VMEM((128,), jnp.int32),
                       pltpu.VMEM((CH, W), jnp.float32),
                       pltpu.VMEM((32, W), jnp.float32),
                       pltpu.VMEM_SHARED((16 * 32, W), jnp.float32)],
    )
    def k(eaug_h, labf_h, osum_h, lab_v, i0, i1, i2, i3, e_v, z_v, acc_sh):
        idxs = [i0, i1, i2, i3]
        c = lax.axis_index("c")
        s = lax.axis_index("s")
        gid = 16 * c + s
        pltpu.sync_copy(labf_h.at[pl.ds(gid * CH, CH)], lab_v)
        pltpu.sync_copy(eaug_h.at[pl.ds(gid * CH, CH)], e_v)

        zero16 = jnp.zeros((16,), jnp.float32)

        for i in range(32):
            for jw in range(W // 16):
                z_v[i, pl.ds(jw * 16, 16)] = zero16
        pltpu.sync_copy(z_v, acc_sh.at[pl.ds(s * 32, 32)])

        s32 = jnp.full((16,), s * 32, jnp.int32)
        for kk in range(4):
            for i in range(8):
                c16 = lab_v[pl.ds(kk * 128 + i * 16, 16)] + s32
                idxs[kk][pl.ds(i * 16, 16)] = c16

        for kk in range(4):
            pltpu.sync_copy(e_v.at[pl.ds(kk * 128, 128)],
                            acc_sh.at[idxs[kk]], add=True)

        pltpu.sync_copy(acc_sh.at[pl.ds(s * 32, 32)],
                        osum_h.at[pl.ds(gid * 32, 32)])

    return k(eaug, labf).reshape(B, 8, 32, W)


def _instance_body(lab_ref, en_ref, ent_ref, s8_ref,
                   pull_ref, push_ref, *, S, N, D):
    first = pl.program_id(0) == 0
    lab = lab_ref[0]
    en = en_ref[0]
    ent = ent_ref[0]

    seg = lax.broadcasted_iota(jnp.int32, (S, N), 0)
    oh = (seg == lab).astype(jnp.float32)
    tot = jnp.sum(s8_ref[0], axis=0)
    sums = tot[:, :D]
    cnt = tot[:, D:D + 1]
    den = jnp.where(cnt > 0, cnt, 1.0)
    means = sums / den

    e2row = jnp.sum(ent * ent, axis=0, keepdims=True)
    x = lax.dot_general(means, ent, (((1,), (0,)), ((), ())),
                        preferred_element_type=jnp.float32, precision=lax.Precision.HIGHEST)
    m2col = jnp.sum(means * means, axis=1, keepdims=True)
    d2 = e2row - 2.0 * x + m2col
    pos = d2 > 1e-12
    d = jnp.where(pos, jnp.sqrt(jnp.where(pos, d2, 1.0)), 0.0)
    d = jnp.maximum(d - _DELTA_V, 0.0)
    segsum = jnp.sum(d * oh, axis=1, keepdims=True)

    rid = lax.broadcasted_iota(jnp.int32, (S, 1), 0)
    valid = (rid >= 1) & (rid < _NUM_IDS)
    prescol = valid & (cnt > 0)
    ipl = jnp.sum(jnp.where(prescol, segsum / den, 0.0))
    ni = jnp.sum(prescol.astype(jnp.float32))
    pull_b = ipl / (ni + 1e-6)

    mn = means / jnp.maximum(jnp.sqrt(m2col), 1e-12)
    mm = lax.dot_general(mn, mn, (((1,), (1,)), ((), ())),
                         preferred_element_type=jnp.float32, precision=lax.Precision.HIGHEST)
    ri = lax.broadcasted_iota(jnp.int32, (S, S), 0)
    ci = lax.broadcasted_iota(jnp.int32, (S, S), 1)
    diag = ri == ci
    mn2row = jnp.sum(jnp.where(diag, mm, 0.0), axis=0, keepdims=True)
    mn2col = jnp.sum(jnp.where(diag, mm, 0.0), axis=1, keepdims=True)
    d2m = mn2row + mn2col - 2.0 * mm
    posm = d2m > 1e-12
    dm = jnp.where(posm, jnp.sqrt(jnp.where(posm, d2m, 1.0)), 0.0)
    presf = prescol.astype(jnp.float32)
    presrow = jnp.sum(jnp.where(diag, presf, 0.0), axis=0, keepdims=True)
    tri = (ci > ri).astype(jnp.float32) * presf * presrow
    hinge = jnp.maximum(2.0 * _DELTA_D - dm, 0.0)
    push_b = jnp.sum(hinge * tri) / (jnp.sum(tri) + 1e-6)

    @pl.when(first)
    def _():
        pull_ref[...] = jnp.zeros((1, 1), jnp.float32)
        push_ref[...] = jnp.zeros((1, 1), jnp.float32)

    pull_ref[...] += jnp.reshape(pull_b, (1, 1))
    push_ref[...] += jnp.reshape(push_b, (1, 1))


def _instance(labels, en, ent, sums8):
    B, N, D = en.shape
    S = 32
    W = sums8.shape[3]
    return pl.pallas_call(
        functools.partial(_instance_body, S=S, N=N, D=D),
        grid=(B,),
        in_specs=[
            pl.BlockSpec((1, 1, N), lambda b: (b, 0, 0)),
            pl.BlockSpec((1, N, D), lambda b: (b, 0, 0)),
            pl.BlockSpec((1, D, N), lambda b: (b, 0, 0)),
            pl.BlockSpec((1, 8, S, W), lambda b: (b, 0, 0, 0)),
        ],
        out_specs=[
            pl.BlockSpec((1, 1), lambda b: (0, 0)),
            pl.BlockSpec((1, 1), lambda b: (0, 0)),
        ],
        out_shape=[
            jax.ShapeDtypeStruct((1, 1), jnp.float32),
            jax.ShapeDtypeStruct((1, 1), jnp.float32),
        ],
    )(labels, en, ent, sums8)


def kernel(points, embeddings, instance_labels):
    B, N, D = embeddings.shape
    pts = points.astype(jnp.float32)
    lab = instance_labels.astype(jnp.int32).reshape(B, 1, N)

    pts_pad = jnp.pad(pts, ((0, 0), (0, 0), (0, 8 - pts.shape[2])))
    s_row = jnp.sum(pts * pts, axis=2)[:, None, :]

    en = _normalize(embeddings.astype(jnp.float32))
    ent = jnp.transpose(en, (0, 2, 1))

    R = 1024
    sm_sum = _smooth(pts_pad, s_row, en, R)[0, 0]
    eaug = jnp.concatenate(
        [en, jnp.ones((B, N, 16), jnp.float32),
         jnp.zeros((B, N, 128 - D - 16), jnp.float32)],
        axis=2).reshape(B * N, 128)
    sums8 = _seg_sums(eaug, lab.reshape(B * N), B, N, 128)
    pull_s, push_s = _instance(lab, en, ent, sums8)

    sm = sm_sum / jnp.float32(N * _K * B)
    pull = pull_s[0, 0] / jnp.float32(B)
    push = push_s[0, 0] / jnp.float32(B)
    inst = pull + push
    total = inst + 0.1 * sm
    return total, inst, sm, pull, push
